# baseline (speedup 1.0000x reference)
import functools
import jax
import jax.numpy as jnp
import numpy as np
from jax.experimental import pallas as pl
from jax.experimental.pallas import tpu as pltpu

_NC = 80
_B, _NT = 16, 32
_ANCHORS = [
    [(0.02, 0.03), (0.04, 0.07), (0.08, 0.06)],
    [(0.07, 0.15), (0.15, 0.11), (0.14, 0.29)],
    [(0.28, 0.22), (0.38, 0.48), (0.90, 0.78)],
]
_HW = [(80, 80), (40, 40), (20, 20)]


def _sigmoid(x):
    return jax.nn.sigmoid(x)


def _softplus_ref(x):
    return jnp.maximum(x, 0.0) + jnp.log1p(jnp.exp(-jnp.abs(x)))


def _focal0(x):
    s = _sigmoid(x)
    return _softplus_ref(x) * s * s


def _focal1(x):
    s = _sigmoid(x)
    return (_softplus_ref(x) - x) * (1.0 - s) * (1.0 - s)


def _build_kernel(t_ref, iout_ref, fout_ref):
    cls_f = t_ref[0]
    xc = t_ref[1]
    yc = t_ref[2]
    w = t_ref[3]
    h = t_ref[4]
    valid = (w > 0.0) & (h > 0.0)
    validf = valid.astype(jnp.float32)
    b_iota = jax.lax.broadcasted_iota(jnp.int32, (_B, _NT), 0)
    for s in range(3):
        H, W = _HW[s]
        anchors = _ANCHORS[s]
        ious = []
        for (aw, ah) in anchors:
            inter = jnp.minimum(w, aw) * jnp.minimum(h, ah)
            ious.append(inter / (w * h + aw * ah - inter + 1e-6))
        best = jnp.zeros((_B, _NT), jnp.int32)
        ibest = ious[0]
        best = jnp.where(ious[1] > ibest, 1, best)
        ibest = jnp.maximum(ibest, ious[1])
        best = jnp.where(ious[2] > ibest, 2, best)
        aw_b = jnp.where(best == 0, anchors[0][0],
                         jnp.where(best == 1, anchors[1][0], anchors[2][0]))
        ah_b = jnp.where(best == 0, anchors[0][1],
                         jnp.where(best == 1, anchors[1][1], anchors[2][1]))
        gi = jnp.clip((xc * W).astype(jnp.int32), 0, W - 1)
        gj = jnp.clip((yc * H).astype(jnp.int32), 0, H - 1)
        flat = ((b_iota * 3 + best) * H + gj) * W + gi
        eq = (flat[:, None, :] == flat[:, :, None])
        later = (jax.lax.broadcasted_iota(jnp.int32, (_NT, _NT), 1) >
                 jax.lax.broadcasted_iota(jnp.int32, (_NT, _NT), 0))[None, :, :]
        kill = eq & later & valid[:, None, :]
        loses = jnp.sum(kill.astype(jnp.float32), axis=2) > 0.0
        winner = validf * (1.0 - loses.astype(jnp.float32))
        npos = jnp.sum(winner)
        tx = xc * W - gi.astype(jnp.float32)
        ty = yc * H - gj.astype(jnp.float32)
        tw = jnp.log(w / aw_b + 1e-6)
        th = jnp.log(h / ah_b + 1e-6)
        iout_ref[s, 0] = best
        iout_ref[s, 1] = gj
        iout_ref[s, 2] = gi
        iout_ref[s, 3] = cls_f.astype(jnp.int32)
        fout_ref[s, 0] = tx
        fout_ref[s, 1] = ty
        fout_ref[s, 2] = tw
        fout_ref[s, 3] = th
        fout_ref[s, 4] = winner
        fout_ref[s, 5] = jnp.full((_B, _NT), npos)


def _sparse_kernel(b_ref, a_ref, gj_ref, gi_ref, cls_ref,
                   pred_ref, f_ref, out_ref, *, W):
    i = pl.program_id(0)

    @pl.when(i == 0)
    def _():
        out_ref[...] = jnp.zeros_like(out_ref)

    p = pred_ref[0, :, 0, 0, :]
    gi = gi_ref[i]
    c = cls_ref[i]
    tx = f_ref[0, 0, 0]
    ty = f_ref[0, 0, 1]
    tw = f_ref[0, 0, 2]
    th = f_ref[0, 0, 3]
    win = f_ref[0, 0, 4]
    mc = (jax.lax.broadcasted_iota(jnp.int32, (1, W), 1) == gi).astype(jnp.float32)

    d0 = _sigmoid(p[0:1, :]) - tx
    d1 = _sigmoid(p[1:2, :]) - ty
    v_xy = jnp.sum((d0 * d0 + d1 * d1) * mc)
    d2 = p[2:3, :] - tw
    d3 = p[3:4, :] - th
    v_wh = jnp.sum((d2 * d2 + d3 * d3) * mc)
    pobj = p[4:5, :]
    v_op = jnp.sum(_focal1(pobj) * mc)
    v_on = jnp.sum(_focal0(pobj) * mc)
    pcls = p[5:85, :]
    tgt = (jax.lax.broadcasted_iota(jnp.int32, (80, W), 0) == c).astype(jnp.float32)
    fc = tgt * _focal1(pcls) + (1.0 - tgt) * _focal0(pcls)
    v_cls = jnp.sum(fc * mc)

    col = jax.lax.broadcasted_iota(jnp.int32, (8, 128), 1)
    row = jax.lax.broadcasted_iota(jnp.int32, (8, 128), 0)
    on_row = (row == 0).astype(jnp.float32)
    upd = (jnp.where(col == 0, v_xy, 0.0) + jnp.where(col == 1, v_wh, 0.0) +
           jnp.where(col == 2, v_op, 0.0) + jnp.where(col == 3, v_on, 0.0) +
           jnp.where(col == 4, v_cls, 0.0)) * on_row * win
    out_ref[...] += upd


def _dense_kernel(pred_ref, out_ref):
    a = pl.program_id(0)
    b = pl.program_id(1)

    @pl.when((a == 0) & (b == 0))
    def _():
        out_ref[...] = jnp.zeros_like(out_ref)

    x = pred_ref[0, 0]
    v = jnp.sum(_focal0(x))
    row = jax.lax.broadcasted_iota(jnp.int32, (8, 128), 0)
    col = jax.lax.broadcasted_iota(jnp.int32, (8, 128), 1)
    out_ref[...] += jnp.where((row == 0) & (col == 0), v, 0.0)


def _run_build(targets):
    t5 = jnp.transpose(targets, (2, 0, 1))
    iout, fout = pl.pallas_call(
        _build_kernel,
        out_shape=(
            jax.ShapeDtypeStruct((3, 4, _B, _NT), jnp.int32),
            jax.ShapeDtypeStruct((3, 6, _B, _NT), jnp.float32),
        ),
    )(t5)
    return iout, fout


def _run_sparse(pred, iout_s, fout_s, H, W):
    pred5 = pred.reshape(_B, 255, H, 1, W)
    n = _B * _NT
    a_arr = iout_s[0].reshape(n)
    gj_arr = iout_s[1].reshape(n)
    gi_arr = iout_s[2].reshape(n)
    cls_arr = iout_s[3].reshape(n)
    b_arr = (jnp.arange(n, dtype=jnp.int32) // _NT)
    f_s = jnp.transpose(fout_s[:5], (1, 2, 0)).reshape(n, 1, 5)
    f_s = jnp.pad(f_s, ((0, 0), (0, 0), (0, 3)))

    grid_spec = pltpu.PrefetchScalarGridSpec(
        num_scalar_prefetch=5,
        grid=(n,),
        in_specs=[
            pl.BlockSpec((1, 85, 1, 1, W),
                         lambda i, b, a, gj, gi, c: (b[i], a[i], gj[i], 0, 0)),
            pl.BlockSpec((1, 1, 8),
                         lambda i, b, a, gj, gi, c: (i, 0, 0)),
        ],
        out_specs=pl.BlockSpec((8, 128),
                               lambda i, b, a, gj, gi, c: (0, 0)),
    )
    out = pl.pallas_call(
        functools.partial(_sparse_kernel, W=W),
        grid_spec=grid_spec,
        out_shape=jax.ShapeDtypeStruct((8, 128), jnp.float32),
        compiler_params=pltpu.CompilerParams(
            dimension_semantics=("arbitrary",)),
    )(b_arr, a_arr, gj_arr, gi_arr, cls_arr, pred5, f_s)
    return out[0, :5]


def _run_dense(pred, H, W):
    out = pl.pallas_call(
        _dense_kernel,
        grid=(3, _B),
        in_specs=[pl.BlockSpec((1, 1, H, W),
                               lambda a, b: (b, a * 85 + 4, 0, 0))],
        out_specs=pl.BlockSpec((8, 128), lambda a, b: (0, 0)),
        out_shape=jax.ShapeDtypeStruct((8, 128), jnp.float32),
        compiler_params=pltpu.CompilerParams(
            dimension_semantics=("arbitrary", "arbitrary")),
    )(pred)
    return out[0, 0]


def kernel(pred_s0, pred_s1, pred_s2, targets):
    preds = [pred_s0, pred_s1, pred_s2]
    iout, fout = _run_build(targets)
    total = jnp.float32(0.0)
    for s in range(3):
        H, W = _HW[s]
        sums = _run_sparse(preds[s], iout[s], fout[s], H, W)
        dense_neg = _run_dense(preds[s], H, W)
        npos_raw = fout[s, 5, 0, 0]
        npos = jnp.maximum(npos_raw, 1.0)
        N = jnp.float32(_B * 3 * H * W)
        nneg = jnp.maximum(N - npos_raw, 1.0)
        v_xy, v_wh, v_op, v_on, v_cls = (sums[0], sums[1], sums[2],
                                         sums[3], sums[4])
        loss_box = (v_xy + v_wh) / (npos * 2.0)
        loss_obj_pos = v_op / npos
        loss_obj_neg = (dense_neg - v_on) / nneg
        loss_cls = v_cls / (npos * _NC)
        total = total + (5.0 * loss_box + loss_obj_pos +
                         0.5 * loss_obj_neg + loss_cls)
    return total / 3.0


# speedup vs baseline: 1.2803x; 1.2803x over previous
import functools
import jax
import jax.numpy as jnp
import numpy as np
from jax.experimental import pallas as pl
from jax.experimental.pallas import tpu as pltpu

_NC = 80
_B, _NT = 16, 32
_ANCHORS = [
    [(0.02, 0.03), (0.04, 0.07), (0.08, 0.06)],
    [(0.07, 0.15), (0.15, 0.11), (0.14, 0.29)],
    [(0.28, 0.22), (0.38, 0.48), (0.90, 0.78)],
]
_HW = [(80, 80), (40, 40), (20, 20)]


def _sigmoid(x):
    return jax.nn.sigmoid(x)


def _softplus_ref(x):
    return jnp.maximum(x, 0.0) + jnp.log1p(jnp.exp(-jnp.abs(x)))


def _focal0(x):
    s = _sigmoid(x)
    return _softplus_ref(x) * s * s


def _focal1(x):
    s = _sigmoid(x)
    return (_softplus_ref(x) - x) * (1.0 - s) * (1.0 - s)


def _build_kernel(t_ref, iout_ref, fout_ref):
    cls_f = t_ref[0]
    xc = t_ref[1]
    yc = t_ref[2]
    w = t_ref[3]
    h = t_ref[4]
    valid = (w > 0.0) & (h > 0.0)
    validf = valid.astype(jnp.float32)
    b_iota = jax.lax.broadcasted_iota(jnp.int32, (_B, _NT), 0)
    for s in range(3):
        H, W = _HW[s]
        anchors = _ANCHORS[s]
        ious = []
        for (aw, ah) in anchors:
            inter = jnp.minimum(w, aw) * jnp.minimum(h, ah)
            ious.append(inter / (w * h + aw * ah - inter + 1e-6))
        best = jnp.zeros((_B, _NT), jnp.int32)
        ibest = ious[0]
        best = jnp.where(ious[1] > ibest, 1, best)
        ibest = jnp.maximum(ibest, ious[1])
        best = jnp.where(ious[2] > ibest, 2, best)
        aw_b = jnp.where(best == 0, anchors[0][0],
                         jnp.where(best == 1, anchors[1][0], anchors[2][0]))
        ah_b = jnp.where(best == 0, anchors[0][1],
                         jnp.where(best == 1, anchors[1][1], anchors[2][1]))
        gi = jnp.clip((xc * W).astype(jnp.int32), 0, W - 1)
        gj = jnp.clip((yc * H).astype(jnp.int32), 0, H - 1)
        flat = ((b_iota * 3 + best) * H + gj) * W + gi
        eq = (flat[:, None, :] == flat[:, :, None])
        later = (jax.lax.broadcasted_iota(jnp.int32, (_NT, _NT), 1) >
                 jax.lax.broadcasted_iota(jnp.int32, (_NT, _NT), 0))[None, :, :]
        kill = eq & later & valid[:, None, :]
        loses = jnp.sum(kill.astype(jnp.float32), axis=2) > 0.0
        winner = validf * (1.0 - loses.astype(jnp.float32))
        npos = jnp.sum(winner)
        tx = xc * W - gi.astype(jnp.float32)
        ty = yc * H - gj.astype(jnp.float32)
        tw = jnp.log(w / aw_b + 1e-6)
        th = jnp.log(h / ah_b + 1e-6)
        iout_ref[s, 0] = best
        iout_ref[s, 1] = gj
        iout_ref[s, 2] = gi
        iout_ref[s, 3] = cls_f.astype(jnp.int32)
        fout_ref[s, 0] = tx
        fout_ref[s, 1] = ty
        fout_ref[s, 2] = tw
        fout_ref[s, 3] = th
        fout_ref[s, 4] = winner
        fout_ref[s, 5] = jnp.full((_B, _NT), npos)


def _scale_kernel(a_ref, gj_ref, gi_ref, cls_ref, *refs, W):
    slabs = refs[:_NT]
    objs = refs[_NT:_NT + 3]
    f_ref = refs[_NT + 3]
    out_ref = refs[_NT + 4]
    b = pl.program_id(0)

    @pl.when(b == 0)
    def _():
        out_ref[...] = jnp.zeros_like(out_ref)

    v_xy = jnp.float32(0.0)
    v_wh = jnp.float32(0.0)
    v_op = jnp.float32(0.0)
    v_on = jnp.float32(0.0)
    v_cls = jnp.float32(0.0)
    lane = jax.lax.broadcasted_iota(jnp.int32, (1, W), 1)
    lane80 = jax.lax.broadcasted_iota(jnp.int32, (80, W), 0)
    for t in range(_NT):
        p = slabs[t][0, :, 0, 0, :]
        gi = gi_ref[b, t]
        c = cls_ref[b, t]
        tx = f_ref[0, t, 0]
        ty = f_ref[0, t, 1]
        tw = f_ref[0, t, 2]
        th = f_ref[0, t, 3]
        win = f_ref[0, t, 4]
        mc = jnp.where(lane == gi, win, 0.0)
        d0 = _sigmoid(p[0:1, :]) - tx
        d1 = _sigmoid(p[1:2, :]) - ty
        v_xy += jnp.sum((d0 * d0 + d1 * d1) * mc)
        d2 = p[2:3, :] - tw
        d3 = p[3:4, :] - th
        v_wh += jnp.sum((d2 * d2 + d3 * d3) * mc)
        pobj = p[4:5, :]
        v_op += jnp.sum(_focal1(pobj) * mc)
        v_on += jnp.sum(_focal0(pobj) * mc)
        pcls = p[5:85, :]
        tgt = (lane80 == c).astype(jnp.float32)
        fc = tgt * _focal1(pcls) + (1.0 - tgt) * _focal0(pcls)
        v_cls += jnp.sum(fc * mc)

    v_dense = jnp.float32(0.0)
    for a in range(3):
        v_dense += jnp.sum(_focal0(objs[a][0, 0]))

    col = jax.lax.broadcasted_iota(jnp.int32, (8, 128), 1)
    row = jax.lax.broadcasted_iota(jnp.int32, (8, 128), 0)
    on_row = (row == 0).astype(jnp.float32)
    upd = (jnp.where(col == 0, v_xy, 0.0) + jnp.where(col == 1, v_wh, 0.0) +
           jnp.where(col == 2, v_op, 0.0) + jnp.where(col == 3, v_on, 0.0) +
           jnp.where(col == 4, v_cls, 0.0) +
           jnp.where(col == 5, v_dense, 0.0)) * on_row
    out_ref[...] += upd


def _run_build(targets):
    t5 = jnp.transpose(targets, (2, 0, 1))
    iout, fout = pl.pallas_call(
        _build_kernel,
        out_shape=(
            jax.ShapeDtypeStruct((3, 4, _B, _NT), jnp.int32),
            jax.ShapeDtypeStruct((3, 6, _B, _NT), jnp.float32),
        ),
    )(t5)
    return iout, fout


def _slab_spec(t, W):
    return pl.BlockSpec(
        (1, 85, 1, 1, W),
        lambda b, a, gj, gi, c, t=t: (b, a[b, t], gj[b, t], 0, 0))


def _obj_spec(ai, H, W):
    return pl.BlockSpec(
        (1, 1, H, W), lambda b, a, gj, gi, c, ai=ai: (b, 85 * ai + 4, 0, 0))


def _run_scale(pred, iout_s, fout_s, H, W):
    pred5 = pred.reshape(_B, 255, H, 1, W)
    a2 = iout_s[0]
    gj2 = iout_s[1]
    gi2 = iout_s[2]
    cls2 = iout_s[3]
    f_s = jnp.transpose(fout_s[:5], (1, 2, 0))
    f_s = jnp.pad(f_s, ((0, 0), (0, 0), (0, 3)))

    in_specs = ([_slab_spec(t, W) for t in range(_NT)] +
                [_obj_spec(ai, H, W) for ai in range(3)] +
                [pl.BlockSpec((1, _NT, 8), lambda b, a, gj, gi, c: (b, 0, 0))])
    grid_spec = pltpu.PrefetchScalarGridSpec(
        num_scalar_prefetch=4,
        grid=(_B,),
        in_specs=in_specs,
        out_specs=pl.BlockSpec((8, 128), lambda b, a, gj, gi, c: (0, 0)),
    )
    operands = [pred5] * _NT + [pred] * 3 + [f_s]
    out = pl.pallas_call(
        functools.partial(_scale_kernel, W=W),
        grid_spec=grid_spec,
        out_shape=jax.ShapeDtypeStruct((8, 128), jnp.float32),
        compiler_params=pltpu.CompilerParams(
            dimension_semantics=("arbitrary",)),
    )(a2, gj2, gi2, cls2, *operands)
    return out[0, :6]


def kernel(pred_s0, pred_s1, pred_s2, targets):
    preds = [pred_s0, pred_s1, pred_s2]
    iout, fout = _run_build(targets)
    total = jnp.float32(0.0)
    for s in range(3):
        H, W = _HW[s]
        sums = _run_scale(preds[s], iout[s], fout[s], H, W)
        npos_raw = fout[s, 5, 0, 0]
        npos = jnp.maximum(npos_raw, 1.0)
        N = jnp.float32(_B * 3 * H * W)
        nneg = jnp.maximum(N - npos_raw, 1.0)
        v_xy, v_wh, v_op, v_on, v_cls, dense_neg = (
            sums[0], sums[1], sums[2], sums[3], sums[4], sums[5])
        loss_box = (v_xy + v_wh) / (npos * 2.0)
        loss_obj_pos = v_op / npos
        loss_obj_neg = (dense_neg - v_on) / nneg
        loss_cls = v_cls / (npos * _NC)
        total = total + (5.0 * loss_box + loss_obj_pos +
                         0.5 * loss_obj_neg + loss_cls)
    return total / 3.0


# speedup vs baseline: 2.4285x; 1.8968x over previous
import functools
import jax
import jax.numpy as jnp
import numpy as np
from jax import lax
from jax.experimental import pallas as pl
from jax.experimental.pallas import tpu as pltpu
from jax.experimental.pallas import tpu_sc as plsc

_NC = 80
_B, _NT = 16, 32
_NTGT = _B * _NT
_NGATH = _NTGT * 85
_NW = 32
_CHUNK = _NGATH // _NW
_NSUB = 17
_SUBW = 80
_ANCHORS = [
    [(0.02, 0.03), (0.04, 0.07), (0.08, 0.06)],
    [(0.07, 0.15), (0.15, 0.11), (0.14, 0.29)],
    [(0.28, 0.22), (0.38, 0.48), (0.90, 0.78)],
]
_HW = [(80, 80), (40, 40), (20, 20)]


def _sigmoid(x):
    return jax.nn.sigmoid(x)


def _softplus_ref(x):
    return jnp.maximum(x, 0.0) + jnp.log1p(jnp.exp(-jnp.abs(x)))


def _focal0(x):
    s = _sigmoid(x)
    return _softplus_ref(x) * s * s


def _focal1(x):
    s = _sigmoid(x)
    return (_softplus_ref(x) - x) * (1.0 - s) * (1.0 - s)


def _build_kernel(t_ref, fout_ref, idx_ref):
    cls_f = t_ref[0]
    xc = t_ref[1]
    yc = t_ref[2]
    w = t_ref[3]
    h = t_ref[4]
    valid = (w > 0.0) & (h > 0.0)
    validf = valid.astype(jnp.float32)
    b_iota = lax.broadcasted_iota(jnp.int32, (_B, _NT), 0)
    c_iota = lax.broadcasted_iota(jnp.int32, (_B, _NT, 85), 2)
    for s in range(3):
        H, W = _HW[s]
        anchors = _ANCHORS[s]
        ious = []
        for (aw, ah) in anchors:
            inter = jnp.minimum(w, aw) * jnp.minimum(h, ah)
            ious.append(inter / (w * h + aw * ah - inter + 1e-6))
        best = jnp.zeros((_B, _NT), jnp.int32)
        ibest = ious[0]
        best = jnp.where(ious[1] > ibest, 1, best)
        ibest = jnp.maximum(ibest, ious[1])
        best = jnp.where(ious[2] > ibest, 2, best)
        aw_b = jnp.where(best == 0, anchors[0][0],
                         jnp.where(best == 1, anchors[1][0], anchors[2][0]))
        ah_b = jnp.where(best == 0, anchors[0][1],
                         jnp.where(best == 1, anchors[1][1], anchors[2][1]))
        gi = jnp.clip((xc * W).astype(jnp.int32), 0, W - 1)
        gj = jnp.clip((yc * H).astype(jnp.int32), 0, H - 1)
        flat = ((b_iota * 3 + best) * H + gj) * W + gi
        eq = (flat[:, None, :] == flat[:, :, None])
        later = (lax.broadcasted_iota(jnp.int32, (_NT, _NT), 1) >
                 lax.broadcasted_iota(jnp.int32, (_NT, _NT), 0))[None, :, :]
        kill = eq & later & valid[:, None, :]
        loses = jnp.sum(kill.astype(jnp.float32), axis=2) > 0.0
        winner = validf * (1.0 - loses.astype(jnp.float32))
        npos = jnp.sum(winner)
        tx = xc * W - gi.astype(jnp.float32)
        ty = yc * H - gj.astype(jnp.float32)
        tw = jnp.log(w / aw_b + 1e-6)
        th = jnp.log(h / ah_b + 1e-6)
        fout_ref[s, 0] = tx
        fout_ref[s, 1] = ty
        fout_ref[s, 2] = tw
        fout_ref[s, 3] = th
        fout_ref[s, 4] = winner
        fout_ref[s, 5] = cls_f
        fout_ref[s, 6] = jnp.full((_B, _NT), npos)
        base = ((b_iota * 255 + best * 85) * H + gj) * W + gi
        fout_ref[s, 7] = (base % 8).astype(jnp.float32)
        idx_ref[s] = (base // 8)[:, :, None] + c_iota * (H * W // 8)


def _run_build(targets):
    t5 = jnp.transpose(targets, (2, 0, 1))
    fout, idx = pl.pallas_call(
        _build_kernel,
        out_shape=(
            jax.ShapeDtypeStruct((3, 8, _B, _NT), jnp.float32),
            jax.ShapeDtypeStruct((3, _B, _NT, 85), jnp.int32),
        ),
    )(t5)
    return fout, idx


def _sc_gather_kernel(idx_hbm, p0_hbm, p1_hbm, p2_hbm, out_hbm,
                      idx_v, rows_v, sem):
    wid = lax.axis_index("s") * 2 + lax.axis_index("c")
    preds = [p0_hbm, p1_hbm, p2_hbm]
    for s in range(3):
        pltpu.sync_copy(idx_hbm.at[s, wid], idx_v.at[s])
    for s in range(3):
        pred = preds[s]

        def step(j, carry, s=s, pred=pred):
            pltpu.async_copy(
                pred.at[idx_v.at[s, j]], rows_v.at[s, j], sem).wait()
            return carry

        lax.fori_loop(0, _NSUB, step, 0)
    for s in range(3):
        pltpu.sync_copy(rows_v.at[s], out_hbm.at[s, wid])


def _run_sc_gather(idx, preds_flat):
    idx_t = idx.reshape(3, _NW, _NSUB, _SUBW)
    mesh = plsc.VectorSubcoreMesh(core_axis_name="c", subcore_axis_name="s",
                                  num_cores=2, num_subcores=16)
    out = pl.kernel(
        _sc_gather_kernel,
        out_type=jax.ShapeDtypeStruct((3, _NW, _NSUB, _SUBW, 8), jnp.float32),
        mesh=mesh,
        scratch_types=[
            pltpu.VMEM((3, _NSUB, _SUBW), jnp.int32),
            pltpu.VMEM((3, _NSUB, _SUBW, 8), jnp.float32),
            pltpu.SemaphoreType.DMA,
        ],
        compiler_params=pltpu.CompilerParams(use_tc_tiling_on_sc=False),
    )(idx_t, *preds_flat)
    return out.reshape(3, _NTGT, 680)


def _loss_kernel(g_ref, f_ref, out_ref):
    g8 = g_ref[0]
    f = f_ref[0]
    o_i = f[:, 7:8].astype(jnp.int32)
    lane680 = lax.broadcasted_iota(jnp.int32, (_NTGT, 680), 1)
    m680 = (lane680 % 8 == o_i).astype(jnp.float32)
    sel = (lax.broadcasted_iota(jnp.int32, (680, 85), 0) // 8 ==
           lax.broadcasted_iota(jnp.int32, (680, 85), 1)).astype(jnp.float32)
    p = jnp.dot(g8 * m680, sel, preferred_element_type=jnp.float32)
    tx = f[:, 0:1]
    ty = f[:, 1:2]
    tw = f[:, 2:3]
    th = f[:, 3:4]
    win = f[:, 4:5]
    c_i = f[:, 5:6].astype(jnp.int32)
    d0 = _sigmoid(p[:, 0:1]) - tx
    d1 = _sigmoid(p[:, 1:2]) - ty
    v_xy = jnp.sum((d0 * d0 + d1 * d1) * win)
    d2 = p[:, 2:3] - tw
    d3 = p[:, 3:4] - th
    v_wh = jnp.sum((d2 * d2 + d3 * d3) * win)
    pobj = p[:, 4:5]
    v_op = jnp.sum(_focal1(pobj) * win)
    v_on = jnp.sum(_focal0(pobj) * win)
    pc = p[:, 5:85]
    tgt = (lax.broadcasted_iota(jnp.int32, (_NTGT, 80), 1) == c_i
           ).astype(jnp.float32)
    fc = tgt * _focal1(pc) + (1.0 - tgt) * _focal0(pc)
    v_cls = jnp.sum(fc * win)
    col = lax.broadcasted_iota(jnp.int32, (8, 128), 1)
    row = lax.broadcasted_iota(jnp.int32, (8, 128), 0)
    on_row = (row == 0).astype(jnp.float32)
    out_ref[0] = (jnp.where(col == 0, v_xy, 0.0) +
                  jnp.where(col == 1, v_wh, 0.0) +
                  jnp.where(col == 2, v_op, 0.0) +
                  jnp.where(col == 3, v_on, 0.0) +
                  jnp.where(col == 4, v_cls, 0.0)) * on_row


def _run_loss(gathered, f_s):
    out = pl.pallas_call(
        _loss_kernel,
        grid=(3,),
        in_specs=[
            pl.BlockSpec((1, _NTGT, 680), lambda s: (s, 0, 0)),
            pl.BlockSpec((1, _NTGT, 8), lambda s: (s, 0, 0)),
        ],
        out_specs=pl.BlockSpec((1, 8, 128), lambda s: (s, 0, 0)),
        out_shape=jax.ShapeDtypeStruct((3, 8, 128), jnp.float32),
        compiler_params=pltpu.CompilerParams(
            dimension_semantics=("arbitrary",)),
    )(gathered, f_s)
    return out[:, 0, :5]


def _dense_kernel(pred_ref, out_ref):
    a = pl.program_id(0)
    b = pl.program_id(1)

    @pl.when((a == 0) & (b == 0))
    def _():
        out_ref[...] = jnp.zeros_like(out_ref)

    x = pred_ref[0, 0]
    v = jnp.sum(_focal0(x))
    row = lax.broadcasted_iota(jnp.int32, (8, 128), 0)
    col = lax.broadcasted_iota(jnp.int32, (8, 128), 1)
    out_ref[...] += jnp.where((row == 0) & (col == 0), v, 0.0)


def _run_dense(pred, H, W):
    out = pl.pallas_call(
        _dense_kernel,
        grid=(3, _B),
        in_specs=[pl.BlockSpec((1, 1, H, W),
                               lambda a, b: (b, a * 85 + 4, 0, 0))],
        out_specs=pl.BlockSpec((8, 128), lambda a, b: (0, 0)),
        out_shape=jax.ShapeDtypeStruct((8, 128), jnp.float32),
        compiler_params=pltpu.CompilerParams(
            dimension_semantics=("arbitrary", "arbitrary")),
    )(pred)
    return out[0, 0]


def _finish(preds, gathered, fout):
    f_s = jnp.transpose(fout, (0, 2, 3, 1)).reshape(3, _NTGT, 8)
    sums = _run_loss(gathered, f_s)
    total = jnp.float32(0.0)
    for s in range(3):
        H, W = _HW[s]
        dense_neg = _run_dense(preds[s], H, W)
        npos_raw = fout[s, 6, 0, 0]
        npos = jnp.maximum(npos_raw, 1.0)
        N = jnp.float32(_B * 3 * H * W)
        nneg = jnp.maximum(N - npos_raw, 1.0)
        v_xy, v_wh, v_op, v_on, v_cls = (sums[s, 0], sums[s, 1], sums[s, 2],
                                         sums[s, 3], sums[s, 4])
        loss_box = (v_xy + v_wh) / (npos * 2.0)
        loss_obj_pos = v_op / npos
        loss_obj_neg = (dense_neg - v_on) / nneg
        loss_cls = v_cls / (npos * _NC)
        total = total + (5.0 * loss_box + loss_obj_pos +
                         0.5 * loss_obj_neg + loss_cls)
    return total / 3.0


def kernel(pred_s0, pred_s1, pred_s2, targets):
    preds = [pred_s0, pred_s1, pred_s2]
    fout, idx = _run_build(targets)
    preds_flat = [p.reshape(-1, 8) for p in preds]
    gathered = _run_sc_gather(idx, preds_flat)
    return _finish(preds, gathered, fout)
